# baseline (speedup 1.0000x reference)
import functools

import jax
import jax.numpy as jnp
from jax import lax
from jax.experimental import pallas as pl
from jax.experimental.pallas import tpu as pltpu
from jax.experimental.pallas import tpu_sc as plsc

_P1T = 256
_EPS = 1e-6


def _tc_body(x_ref, y_ref, xn_ref, yn_ref,
             sumx_ref, sumy_ref, idxx_ref, idxy_ref, xnh_ref, ynh_ref,
             ytb_ref, y2_ref, colmin_ref, colarg_ref):
    n = pl.program_id(0)
    i = pl.program_id(1)
    ni = pl.num_programs(1)

    xb = x_ref[0]
    p1t = xb.shape[0]
    p2 = y_ref.shape[1]
    p1 = ni * p1t

    @pl.when(i == 0)
    def _():
        yb = y_ref[0]
        ytb_ref[...] = jnp.transpose(yb, (1, 0)).astype(jnp.bfloat16)
        y2c = jnp.sum(yb * yb, axis=1, keepdims=True)
        y2_ref[...] = jnp.transpose(y2c, (1, 0))
        b = yn_ref[0]
        nb = jnp.maximum(jnp.sqrt(jnp.sum(b * b, axis=1, keepdims=True)),
                         _EPS)
        ynh_ref[0] = b / nb

    x2 = jnp.sum(xb * xb, axis=1, keepdims=True)
    dot = jax.lax.dot_general(
        xb.astype(jnp.bfloat16), ytb_ref[...],
        (((1,), (0,)), ((), ())),
        preferred_element_type=jnp.float32)
    d = (x2 + y2_ref[...]) - 2.0 * dot

    rmin = jnp.min(d, axis=1, keepdims=True)
    it_l = lax.broadcasted_iota(jnp.int32, (p1t, p2), 1)
    rarg = jnp.min(jnp.where(d == rmin, it_l, p2), axis=1, keepdims=True)
    idxx_ref[0] = rarg

    @pl.when((n == 0) & (i == 0))
    def _():
        sumx_ref[0, 0] = 0.0
        sumy_ref[0, 0] = 0.0

    sumx_ref[0, 0] += jnp.sum(rmin)

    cmin = jnp.min(d, axis=0, keepdims=True)
    it_s = lax.broadcasted_iota(jnp.int32, (p1t, p2), 0) + i * p1t
    carg = jnp.min(jnp.where(d == cmin, it_s, p1), axis=0, keepdims=True)

    @pl.when(i == 0)
    def _():
        colmin_ref[...] = cmin
        colarg_ref[...] = carg

    @pl.when(i > 0)
    def _():
        upd = cmin < colmin_ref[...]
        colarg_ref[...] = jnp.where(upd, carg, colarg_ref[...])
        colmin_ref[...] = jnp.where(upd, cmin, colmin_ref[...])

    @pl.when(i == ni - 1)
    def _():
        idxy_ref[0] = colarg_ref[...]
        sumy_ref[0, 0] += jnp.sum(colmin_ref[...])

    a = xn_ref[0]
    na = jnp.maximum(jnp.sqrt(jnp.sum(a * a, axis=1, keepdims=True)), _EPS)
    xnh_ref[0] = a / na


def _tc_stage(x, y, x_normals, y_normals):
    n, p1, _ = x.shape
    p2 = y.shape[1]
    ni = p1 // _P1T
    grid = (n, ni)
    out_shapes = (
        jax.ShapeDtypeStruct((1, 1), jnp.float32),
        jax.ShapeDtypeStruct((1, 1), jnp.float32),
        jax.ShapeDtypeStruct((n, p1, 1), jnp.int32),
        jax.ShapeDtypeStruct((n, 1, p2), jnp.int32),
        jax.ShapeDtypeStruct((n, p1, 3), jnp.float32),
        jax.ShapeDtypeStruct((n, p2, 3), jnp.float32),
    )
    in_specs = [
        pl.BlockSpec((1, _P1T, 3), lambda n_, i: (n_, i, 0)),
        pl.BlockSpec((1, p2, 3), lambda n_, i: (n_, 0, 0)),
        pl.BlockSpec((1, _P1T, 3), lambda n_, i: (n_, i, 0)),
        pl.BlockSpec((1, p2, 3), lambda n_, i: (n_, 0, 0)),
    ]
    out_specs = (
        pl.BlockSpec((1, 1), lambda n_, i: (0, 0), memory_space=pltpu.SMEM),
        pl.BlockSpec((1, 1), lambda n_, i: (0, 0), memory_space=pltpu.SMEM),
        pl.BlockSpec((1, _P1T, 1), lambda n_, i: (n_, i, 0)),
        pl.BlockSpec((1, 1, p2), lambda n_, i: (n_, 0, 0)),
        pl.BlockSpec((1, _P1T, 3), lambda n_, i: (n_, i, 0)),
        pl.BlockSpec((1, p2, 3), lambda n_, i: (n_, 0, 0)),
    )
    return pl.pallas_call(
        _tc_body,
        grid=grid,
        in_specs=in_specs,
        out_specs=out_specs,
        out_shape=out_shapes,
        scratch_shapes=[
            pltpu.VMEM((3, p2), jnp.bfloat16),
            pltpu.VMEM((1, p2), jnp.float32),
            pltpu.VMEM((1, p2), jnp.float32),
            pltpu.VMEM((1, p2), jnp.int32),
        ],
    )(x, y, x_normals, y_normals)


def _sc_normal_loss(xnh, ynh, idx_x, idx_y):
    n, p, _ = xnh.shape
    nwork = 32
    per_b = nwork // n
    chunk = p // per_b
    groups = chunk // 16
    mesh = plsc.VectorSubcoreMesh(core_axis_name="c", subcore_axis_name="s")

    @functools.partial(
        pl.kernel,
        mesh=mesh,
        compiler_params=pltpu.CompilerParams(use_tc_tiling_on_sc=False,
                                             needs_layout_passes=False),
        out_type=jax.ShapeDtypeStruct((2, nwork, 16), jnp.float32),
        scratch_types=[
            pltpu.VMEM((p, 3), jnp.float32),
            pltpu.VMEM((p, 3), jnp.float32),
            pltpu.VMEM((chunk, 1), jnp.int32),
            pltpu.VMEM((chunk,), jnp.int32),
            pltpu.VMEM((16,), jnp.float32),
            pltpu.VMEM((16,), jnp.float32),
        ],
    )
    def k(xnh_hbm, ynh_hbm, idxx_hbm, idxy_hbm, out_hbm,
          xn_v, yn_v, ix_v, iy_v, accx_v, accy_v):
        wid = lax.axis_index("s") * 2 + lax.axis_index("c")
        b = wid // per_b
        q = wid % per_b
        base = q * chunk
        pltpu.sync_copy(xnh_hbm.at[b], xn_v)
        pltpu.sync_copy(ynh_hbm.at[b], yn_v)
        pltpu.sync_copy(idxx_hbm.at[b, pl.ds(base, chunk)], ix_v)
        pltpu.sync_copy(idxy_hbm.at[b, 0, pl.ds(base, chunk)], iy_v)

        lane = lax.iota(jnp.int32, 16)
        c0 = jnp.zeros((16,), jnp.int32)
        c1 = c0 + 1
        c2 = c0 + 2

        def body(g, acc):
            accx, accy = acc
            own = base + g * 16 + lane
            ix = plsc.load_gather(ix_v, [g * 16 + lane, c0])
            a0 = plsc.load_gather(xn_v, [own, c0])
            a1 = plsc.load_gather(xn_v, [own, c1])
            a2 = plsc.load_gather(xn_v, [own, c2])
            g0 = plsc.load_gather(yn_v, [ix, c0])
            g1 = plsc.load_gather(yn_v, [ix, c1])
            g2 = plsc.load_gather(yn_v, [ix, c2])
            cosx = a0 * g0 + a1 * g1 + a2 * g2
            iy = iy_v[pl.ds(g * 16, 16)]
            b0 = plsc.load_gather(yn_v, [own, c0])
            b1 = plsc.load_gather(yn_v, [own, c1])
            b2 = plsc.load_gather(yn_v, [own, c2])
            h0 = plsc.load_gather(xn_v, [iy, c0])
            h1 = plsc.load_gather(xn_v, [iy, c1])
            h2 = plsc.load_gather(xn_v, [iy, c2])
            cosy = b0 * h0 + b1 * h1 + b2 * h2
            return (accx + (1.0 - jnp.abs(cosx)),
                    accy + (1.0 - jnp.abs(cosy)))

        zero = jnp.zeros((16,), jnp.float32)
        accx, accy = lax.fori_loop(0, groups, body, (zero, zero))
        accx_v[...] = accx
        accy_v[...] = accy
        pltpu.sync_copy(accx_v, out_hbm.at[0, wid])
        pltpu.sync_copy(accy_v, out_hbm.at[1, wid])

    return k(xnh, ynh, idx_x, idx_y)


def kernel(x, y, x_normals, y_normals):
    n, p1, _ = x.shape
    p2 = y.shape[1]
    sumx, sumy, idx_x, idx_y, xnh, ynh = _tc_stage(x, y, x_normals, y_normals)
    acc = _sc_normal_loss(xnh, ynh, idx_x, idx_y)
    cham_dist = sumx[0, 0] / (p1 * n) + sumy[0, 0] / (p2 * n)
    cham_normals = (jnp.sum(acc[0]) / (p1 * n)
                    + jnp.sum(acc[1]) / (p2 * n))
    return (cham_dist, cham_normals)


# speedup vs baseline: 1.7418x; 1.7418x over previous
import functools

import jax
import jax.numpy as jnp
from jax import lax
from jax.experimental import pallas as pl
from jax.experimental.pallas import tpu as pltpu
from jax.experimental.pallas import tpu_sc as plsc

_P1T = 2048
_EPS = 1e-6


def _tc_body(x_ref, yt_ref, xnt_ref, ynt_ref,
             sumx_ref, sumy_ref, idxx_ref, idxy_ref, xnh_ref, ynh_ref,
             colmin_ref, colarg_ref):
    n = pl.program_id(0)
    i = pl.program_id(1)
    ni = pl.num_programs(1)

    xb = x_ref[0]
    yt = yt_ref[0]
    p1t = xb.shape[0]
    p2 = yt.shape[1]
    p1 = ni * p1t

    x2 = jnp.sum(xb * xb, axis=1, keepdims=True)
    y2 = jnp.sum(yt * yt, axis=0, keepdims=True)
    dot = jax.lax.dot_general(
        xb.astype(jnp.bfloat16), yt.astype(jnp.bfloat16),
        (((1,), (0,)), ((), ())),
        preferred_element_type=jnp.float32)
    d = (x2 + y2) - 2.0 * dot

    rmin = jnp.min(d, axis=1, keepdims=True)
    it_l = lax.broadcasted_iota(jnp.int32, (p1t, p2), 1)
    rarg = jnp.min(jnp.where(d == rmin, it_l, p2), axis=1, keepdims=True)
    idxx_ref[0, 0] = rarg

    @pl.when((n == 0) & (i == 0))
    def _():
        sumx_ref[0, 0] = 0.0
        sumy_ref[0, 0] = 0.0

    sumx_ref[0, 0] += jnp.sum(rmin)

    cmin = jnp.min(d, axis=0, keepdims=True)
    it_s = lax.broadcasted_iota(jnp.int32, (p1t, p2), 0) + i * p1t
    carg = jnp.min(jnp.where(d == cmin, it_s, p1), axis=0, keepdims=True)

    @pl.when(i == 0)
    def _():
        colmin_ref[...] = cmin
        colarg_ref[...] = carg

    @pl.when(i > 0)
    def _():
        upd = cmin < colmin_ref[...]
        colarg_ref[...] = jnp.where(upd, carg, colarg_ref[...])
        colmin_ref[...] = jnp.where(upd, cmin, colmin_ref[...])

    @pl.when(i == ni - 1)
    def _():
        idxy_ref[0] = colarg_ref[...]
        sumy_ref[0, 0] += jnp.sum(colmin_ref[...])

    a = xnt_ref[0]
    na = jnp.maximum(jnp.sqrt(jnp.sum(a * a, axis=0, keepdims=True)), _EPS)
    xnh_ref[0] = a / na

    @pl.when(i == 0)
    def _():
        b = ynt_ref[0]
        nb = jnp.maximum(jnp.sqrt(jnp.sum(b * b, axis=0, keepdims=True)), _EPS)
        ynh_ref[0] = b / nb


def _tc_stage(x, y_t, xn_t, yn_t):
    n, p1, _ = x.shape
    p2 = y_t.shape[2]
    ni = p1 // _P1T
    grid = (n, ni)
    out_shapes = (
        jax.ShapeDtypeStruct((1, 1), jnp.float32),
        jax.ShapeDtypeStruct((1, 1), jnp.float32),
        jax.ShapeDtypeStruct((n, ni, _P1T, 1), jnp.int32),
        jax.ShapeDtypeStruct((n, 1, p2), jnp.int32),
        jax.ShapeDtypeStruct((n, 3, p1), jnp.float32),
        jax.ShapeDtypeStruct((n, 3, p2), jnp.float32),
    )
    in_specs = [
        pl.BlockSpec((1, _P1T, 3), lambda n_, i: (n_, i, 0)),
        pl.BlockSpec((1, 3, p2), lambda n_, i: (n_, 0, 0)),
        pl.BlockSpec((1, 3, _P1T), lambda n_, i: (n_, 0, i)),
        pl.BlockSpec((1, 3, p2), lambda n_, i: (n_, 0, 0)),
    ]
    out_specs = (
        pl.BlockSpec((1, 1), lambda n_, i: (0, 0), memory_space=pltpu.SMEM),
        pl.BlockSpec((1, 1), lambda n_, i: (0, 0), memory_space=pltpu.SMEM),
        pl.BlockSpec((1, 1, _P1T, 1), lambda n_, i: (n_, i, 0, 0)),
        pl.BlockSpec((1, 1, p2), lambda n_, i: (n_, 0, 0)),
        pl.BlockSpec((1, 3, _P1T), lambda n_, i: (n_, 0, i)),
        pl.BlockSpec((1, 3, p2), lambda n_, i: (n_, 0, 0)),
    )
    return pl.pallas_call(
        _tc_body,
        grid=grid,
        in_specs=in_specs,
        out_specs=out_specs,
        out_shape=out_shapes,
        scratch_shapes=[
            pltpu.VMEM((1, p2), jnp.float32),
            pltpu.VMEM((1, p2), jnp.int32),
        ],
    )(x, y_t, xn_t, yn_t)


def _sc_normal_loss(xnh_flat, ynh_flat, idx_x, idx_y):
    n, w = xnh_flat.shape
    p = w // 3
    nwork = 32
    per_b = nwork // n
    chunk = p // per_b
    groups = chunk // 16
    mesh = plsc.VectorSubcoreMesh(core_axis_name="c", subcore_axis_name="s")

    @functools.partial(
        pl.kernel,
        mesh=mesh,
        compiler_params=pltpu.CompilerParams(use_tc_tiling_on_sc=False,
                                             needs_layout_passes=False),
        out_type=jax.ShapeDtypeStruct((2, nwork, 16), jnp.float32),
        scratch_types=[
            pltpu.VMEM((w,), jnp.float32),
            pltpu.VMEM((w,), jnp.float32),
            pltpu.VMEM((chunk,), jnp.int32),
            pltpu.VMEM((chunk,), jnp.int32),
            pltpu.VMEM((16,), jnp.float32),
            pltpu.VMEM((16,), jnp.float32),
        ],
    )
    def k(xnh_hbm, ynh_hbm, idxx_hbm, idxy_hbm, out_hbm,
          xn_v, yn_v, ix_v, iy_v, accx_v, accy_v):
        wid = lax.axis_index("s") * 2 + lax.axis_index("c")
        b = wid // per_b
        q = wid % per_b
        base = q * chunk
        pltpu.sync_copy(xnh_hbm.at[b], xn_v)
        pltpu.sync_copy(ynh_hbm.at[b], yn_v)
        pltpu.sync_copy(idxx_hbm.at[b, pl.ds(base, chunk)], ix_v)
        pltpu.sync_copy(idxy_hbm.at[b, pl.ds(base, chunk)], iy_v)

        def body(g, acc):
            accx, accy = acc
            o = g * 16
            ix = ix_v[pl.ds(o, 16)]
            a0 = xn_v[pl.ds(base + o, 16)]
            a1 = xn_v[pl.ds(p + base + o, 16)]
            a2 = xn_v[pl.ds(2 * p + base + o, 16)]
            g0 = plsc.load_gather(yn_v, [ix])
            g1 = plsc.load_gather(yn_v, [ix + p])
            g2 = plsc.load_gather(yn_v, [ix + 2 * p])
            cosx = a0 * g0 + a1 * g1 + a2 * g2
            iy = iy_v[pl.ds(o, 16)]
            b0 = yn_v[pl.ds(base + o, 16)]
            b1 = yn_v[pl.ds(p + base + o, 16)]
            b2 = yn_v[pl.ds(2 * p + base + o, 16)]
            h0 = plsc.load_gather(xn_v, [iy])
            h1 = plsc.load_gather(xn_v, [iy + p])
            h2 = plsc.load_gather(xn_v, [iy + 2 * p])
            cosy = b0 * h0 + b1 * h1 + b2 * h2
            return (accx + (1.0 - jnp.abs(cosx)),
                    accy + (1.0 - jnp.abs(cosy)))

        zero = jnp.zeros((16,), jnp.float32)
        accx, accy = lax.fori_loop(0, groups, body, (zero, zero))
        accx_v[...] = accx
        accy_v[...] = accy
        pltpu.sync_copy(accx_v, out_hbm.at[0, wid])
        pltpu.sync_copy(accy_v, out_hbm.at[1, wid])

    return k(xnh_flat, ynh_flat, idx_x, idx_y)


def kernel(x, y, x_normals, y_normals):
    n, p1, _ = x.shape
    p2 = y.shape[1]
    y_t = jnp.transpose(y, (0, 2, 1))
    xn_t = jnp.transpose(x_normals, (0, 2, 1))
    yn_t = jnp.transpose(y_normals, (0, 2, 1))

    sumx, sumy, idx_x4, idx_y3, xnh, ynh = _tc_stage(x, y_t, xn_t, yn_t)
    idx_x = idx_x4.reshape(n, p1)
    idx_y = idx_y3.reshape(n, p2)

    acc = _sc_normal_loss(xnh.reshape(n, 3 * p1), ynh.reshape(n, 3 * p2),
                          idx_x, idx_y)

    cham_dist = sumx[0, 0] / (p1 * n) + sumy[0, 0] / (p2 * n)
    cham_normals = (jnp.sum(acc[0]) / (p1 * n)
                    + jnp.sum(acc[1]) / (p2 * n))
    return (cham_dist, cham_normals)
